# baseline (speedup 1.0000x reference)
import functools

import jax
import jax.numpy as jnp
from jax.experimental import pallas as pl
from jax.experimental.pallas import tpu as pltpu
from jax.experimental.pallas import tpu_sc as plsc

EMBED_DIM = 32
N_EMBED = 1024
N_ROWS = 16384
BM = 8192
CHUNK = 2048

_SC_CORES = 2
_SC_SUBCORES = 16
_SC_NW = _SC_CORES * _SC_SUBCORES
_ROWS_PER_W = N_ROWS // _SC_NW


def _vq_kernel(x_ref, embed_ref, codest_ref, idx_ref):
    embed = embed_ref[...]
    e2 = jnp.sum(embed * embed, axis=0, keepdims=True)
    lanes = jax.lax.broadcasted_iota(jnp.int32, (CHUNK, N_EMBED), 1)
    rows = jax.lax.broadcasted_iota(jnp.int32, (EMBED_DIM, EMBED_DIM), 0)
    eye = (rows == rows.T).astype(jnp.float32)
    for c in range(BM // CHUNK):
        sl = pl.ds(c * CHUNK, CHUNK)
        x = x_ref[sl, :]
        x2 = jnp.sum(x * x, axis=1, keepdims=True)
        xe = jnp.dot(x, embed, preferred_element_type=jnp.float32)
        d = (x2 - 2.0 * xe) + e2
        idx = jnp.argmin(d, axis=1).astype(jnp.int32)
        xt = jax.lax.dot_general(
            eye, x, dimension_numbers=(((1,), (1,)), ((), ())),
            preferred_element_type=jnp.float32)
        enc = (lanes == idx[:, None]).astype(jnp.float32)
        qt = jax.lax.dot_general(
            embed, enc, dimension_numbers=(((1,), (1,)), ((), ())),
            preferred_element_type=jnp.float32)
        codest_ref[:EMBED_DIM, sl] = xt
        codest_ref[EMBED_DIM:, sl] = qt
        idx_ref[0, 0, sl] = idx


@functools.partial(
    pl.kernel,
    out_type=jax.ShapeDtypeStruct((N_ROWS, EMBED_DIM), jnp.float32),
    mesh=plsc.VectorSubcoreMesh(
        core_axis_name="c", subcore_axis_name="s",
        num_cores=_SC_CORES, num_subcores=_SC_SUBCORES),
    scratch_types=[
        pltpu.VMEM((_ROWS_PER_W,), jnp.int32),
        pltpu.VMEM((_ROWS_PER_W, EMBED_DIM), jnp.float32),
        pltpu.SemaphoreType.DMA,
    ],
    compiler_params=pltpu.CompilerParams(use_tc_tiling_on_sc=False),
)
def _sc_gather(embt_hbm, idx_hbm, out_hbm, idx_v, rows_v, sem):
    wid = jax.lax.axis_index("s") * _SC_CORES + jax.lax.axis_index("c")
    base = wid * _ROWS_PER_W
    pltpu.sync_copy(idx_hbm.at[pl.ds(base, _ROWS_PER_W)], idx_v)
    pltpu.async_copy(embt_hbm.at[idx_v], rows_v, sem).wait()
    pltpu.sync_copy(rows_v, out_hbm.at[pl.ds(base, _ROWS_PER_W)])


@jax.jit
def kernel(inputs, embed):
    lead_shape = inputs.shape[:-1]
    flat = inputs.reshape(-1, EMBED_DIM)
    n = flat.shape[0]
    nblk = n // BM
    codest, idx3 = pl.pallas_call(
        _vq_kernel,
        grid=(nblk,),
        in_specs=[
            pl.BlockSpec((BM, EMBED_DIM), lambda i: (i, 0)),
            pl.BlockSpec((EMBED_DIM, N_EMBED), lambda i: (0, 0)),
        ],
        out_specs=[
            pl.BlockSpec((2 * EMBED_DIM, BM), lambda i: (0, i)),
            pl.BlockSpec((1, 1, BM), lambda i: (i, 0, 0)),
        ],
        out_shape=[
            jax.ShapeDtypeStruct((2 * EMBED_DIM, n), jnp.float32),
            jax.ShapeDtypeStruct((nblk, 1, BM), jnp.int32),
        ],
    )(flat, embed)
    codes_out = codest.T.reshape(*lead_shape, 2 * EMBED_DIM)
    indices = idx3.reshape(lead_shape)
    quantized_st = _sc_gather(embed.T, idx3.reshape(-1)).reshape(
        *lead_shape, EMBED_DIM)
    return (quantized_st, codes_out, indices)
